# initial kernel scaffold (unmeasured)
import jax
import jax.numpy as jnp
from jax import lax
from jax.experimental import pallas as pl
from jax.experimental.pallas import tpu as pltpu

N_DEV = 4
SQ = 512
SKV = 2048
D = 1024
H_GLOBAL = 32
H_PER = 8
DH = 128
SCALE = 0.08838834764831843
CHUNK = SQ // N_DEV


def kernel(x, Wq, Wo, K_ext, V_ext):
    xm = x.reshape(SQ, D)
    K = K_ext.reshape(SKV, H_GLOBAL, DH)
    V = V_ext.reshape(SKV, H_GLOBAL, DH)

    def body(x_ref, wq_ref, wo_ref, k_hbm, v_hbm, out_ref,
             q_ref, attn_ref, k_buf, v_buf, rs_buf,
             kv_sems, rs_send, rs_recv, ag_send, ag_recv):
        my = lax.axis_index("i")
        left = lax.rem(my + N_DEV - 1, N_DEV)
        right = lax.rem(my + 1, N_DEV)
        h0 = my * H_PER

        def start_kv(h):
            slot = h % 2
            kc = pltpu.make_async_copy(
                k_hbm.at[:, h0 + h, :], k_buf.at[slot], kv_sems.at[slot, 0])
            vc = pltpu.make_async_copy(
                v_hbm.at[:, h0 + h, :], v_buf.at[slot], kv_sems.at[slot, 1])
            kc.start()
            vc.start()
            return kc, vc

        pending = start_kv(0)

        q_ref[:, :] = jnp.dot(x_ref[:, :], wq_ref[:, :],
                              preferred_element_type=jnp.float32)

        for h in range(H_PER):
            slot = h % 2
            kc, vc = pending
            kc.wait()
            vc.wait()
            if h + 1 < H_PER:
                pending = start_kv(h + 1)
            q_h = q_ref[:, h * DH:(h + 1) * DH]
            s = lax.dot_general(
                q_h, k_buf[slot],
                (((1,), (1,)), ((), ())),
                preferred_element_type=jnp.float32,
            ) * SCALE
            m = jnp.max(s, axis=1, keepdims=True)
            p = jnp.exp(s - m)
            l = jnp.sum(p, axis=1, keepdims=True)
            o = jnp.dot(p, v_buf[slot], preferred_element_type=jnp.float32)
            attn_ref[:, h * DH:(h + 1) * DH] = o / l

        out_ref[:, :] = jnp.dot(attn_ref[:, :], wo_ref[:, :],
                                preferred_element_type=jnp.float32)

        barrier = pltpu.get_barrier_semaphore()
        for nbr in (left, right):
            pl.semaphore_signal(barrier, inc=1, device_id=(nbr,),
                                device_id_type=pl.DeviceIdType.MESH)
        pl.semaphore_wait(barrier, 2)

        for s in range(N_DEV - 1):
            send_c = lax.rem(my - s + N_DEV, N_DEV)
            recv_c = lax.rem(my - s - 1 + N_DEV, N_DEV)
            rdma = pltpu.make_async_remote_copy(
                src_ref=out_ref.at[pl.ds(send_c * CHUNK, CHUNK), :],
                dst_ref=rs_buf.at[s],
                send_sem=rs_send.at[s],
                recv_sem=rs_recv.at[s],
                device_id=(right,),
                device_id_type=pl.DeviceIdType.MESH,
            )
            rdma.start()
            rdma.wait()
            rows = pl.ds(recv_c * CHUNK, CHUNK)
            out_ref[rows, :] = out_ref[rows, :] + rs_buf[s]

        for s in range(N_DEV - 1):
            c = lax.rem(my + 1 - s + N_DEV, N_DEV)
            rows = pl.ds(c * CHUNK, CHUNK)
            rdma = pltpu.make_async_remote_copy(
                src_ref=out_ref.at[rows, :],
                dst_ref=out_ref.at[rows, :],
                send_sem=ag_send.at[s],
                recv_sem=ag_recv.at[s],
                device_id=(right,),
                device_id_type=pl.DeviceIdType.MESH,
            )
            rdma.start()
            rdma.wait()

    out = pl.pallas_call(
        body,
        out_shape=jax.ShapeDtypeStruct((SQ, D), jnp.float32),
        in_specs=[
            pl.BlockSpec(memory_space=pltpu.VMEM),
            pl.BlockSpec(memory_space=pltpu.VMEM),
            pl.BlockSpec(memory_space=pltpu.VMEM),
            pl.BlockSpec(memory_space=pltpu.ANY),
            pl.BlockSpec(memory_space=pltpu.ANY),
        ],
        out_specs=pl.BlockSpec(memory_space=pltpu.VMEM),
        scratch_shapes=[
            pltpu.VMEM((SQ, D), jnp.float32),
            pltpu.VMEM((SQ, D), jnp.float32),
            pltpu.VMEM((2, SKV, DH), jnp.float32),
            pltpu.VMEM((2, SKV, DH), jnp.float32),
            pltpu.VMEM((N_DEV - 1, CHUNK, D), jnp.float32),
            pltpu.SemaphoreType.DMA((2, 2)),
            pltpu.SemaphoreType.DMA((N_DEV - 1,)),
            pltpu.SemaphoreType.DMA((N_DEV - 1,)),
            pltpu.SemaphoreType.DMA((N_DEV - 1,)),
            pltpu.SemaphoreType.DMA((N_DEV - 1,)),
        ],
        compiler_params=pltpu.CompilerParams(collective_id=0),
    )(xm, Wq, Wo, K, V)
    return out.reshape(1, SQ, D)


# baseline (device time: 82465 ns/iter reference)
import jax
import jax.numpy as jnp
from jax import lax
from jax.experimental import pallas as pl
from jax.experimental.pallas import tpu as pltpu

N_DEV = 4
SQ = 512
SKV = 2048
D = 1024
H_GLOBAL = 32
H_PER = 8
DH = 128
SCALE = 0.08838834764831843
CHUNK = SQ // N_DEV


def kernel(x, Wq, Wo, K_ext, V_ext):
    xm = x.reshape(SQ, D)
    K = K_ext.reshape(SKV, H_GLOBAL, DH)
    V = V_ext.reshape(SKV, H_GLOBAL, DH)

    def body(x_ref, wq_ref, wo_ref, k_hbm, v_hbm, out_ref,
             q_ref, attn_ref, k_buf, v_buf, rs_buf,
             kv_sems, rs_send, rs_recv, ag_send, ag_recv):
        my = lax.axis_index("i")
        left = lax.rem(my + N_DEV - 1, N_DEV)
        right = lax.rem(my + 1, N_DEV)
        h0 = my * H_PER

        def start_kv(h):
            slot = h % 2
            kc = pltpu.make_async_copy(
                k_hbm.at[:, h0 + h, :], k_buf.at[slot], kv_sems.at[slot, 0])
            vc = pltpu.make_async_copy(
                v_hbm.at[:, h0 + h, :], v_buf.at[slot], kv_sems.at[slot, 1])
            kc.start()
            vc.start()
            return kc, vc

        pending = start_kv(0)

        q_ref[:, :] = jnp.dot(x_ref[:, :], wq_ref[:, :],
                              preferred_element_type=jnp.float32)

        for h in range(H_PER):
            slot = h % 2
            kc, vc = pending
            kc.wait()
            vc.wait()
            if h + 1 < H_PER:
                pending = start_kv(h + 1)
            q_h = q_ref[:, h * DH:(h + 1) * DH]
            s = lax.dot_general(
                q_h, k_buf[slot],
                (((1,), (1,)), ((), ())),
                preferred_element_type=jnp.float32,
            ) * SCALE
            m = jnp.max(s, axis=1, keepdims=True)
            p = jnp.exp(s - m)
            l = jnp.sum(p, axis=1, keepdims=True)
            o = jnp.dot(p, v_buf[slot], preferred_element_type=jnp.float32)
            attn_ref[:, h * DH:(h + 1) * DH] = o / l

        out_ref[:, :] = jnp.dot(attn_ref[:, :], wo_ref[:, :],
                                preferred_element_type=jnp.float32)

        barrier = pltpu.get_barrier_semaphore()
        for nbr in (left, right):
            pl.semaphore_signal(barrier, inc=1, device_id=(nbr,),
                                device_id_type=pl.DeviceIdType.MESH)
        pl.semaphore_wait(barrier, 2)

        for s in range(N_DEV - 1):
            send_c = lax.rem(my - s + N_DEV, N_DEV)
            recv_c = lax.rem(my - s - 1 + N_DEV, N_DEV)
            rdma = pltpu.make_async_remote_copy(
                src_ref=out_ref.at[pl.ds(send_c * CHUNK, CHUNK), :],
                dst_ref=rs_buf.at[s],
                send_sem=rs_send.at[s],
                recv_sem=rs_recv.at[s],
                device_id=(right,),
                device_id_type=pl.DeviceIdType.MESH,
            )
            rdma.start()
            rdma.wait()
            rows = pl.ds(recv_c * CHUNK, CHUNK)
            out_ref[rows, :] = out_ref[rows, :] + rs_buf[s]

        for s in range(N_DEV - 1):
            c = lax.rem(my + 1 - s + N_DEV, N_DEV)
            rows = pl.ds(c * CHUNK, CHUNK)
            rdma = pltpu.make_async_remote_copy(
                src_ref=out_ref.at[rows, :],
                dst_ref=out_ref.at[rows, :],
                send_sem=ag_send.at[s],
                recv_sem=ag_recv.at[s],
                device_id=(right,),
                device_id_type=pl.DeviceIdType.MESH,
            )
            rdma.start()
            rdma.wait()

    out = pl.pallas_call(
        body,
        out_shape=jax.ShapeDtypeStruct((SQ, D), jnp.float32),
        in_specs=[
            pl.BlockSpec(memory_space=pltpu.VMEM),
            pl.BlockSpec(memory_space=pltpu.VMEM),
            pl.BlockSpec(memory_space=pltpu.VMEM),
            pl.BlockSpec(memory_space=pltpu.MemorySpace.HBM),
            pl.BlockSpec(memory_space=pltpu.MemorySpace.HBM),
        ],
        out_specs=pl.BlockSpec(memory_space=pltpu.VMEM),
        scratch_shapes=[
            pltpu.VMEM((SQ, D), jnp.float32),
            pltpu.VMEM((SQ, D), jnp.float32),
            pltpu.VMEM((2, SKV, DH), jnp.float32),
            pltpu.VMEM((2, SKV, DH), jnp.float32),
            pltpu.VMEM((N_DEV - 1, CHUNK, D), jnp.float32),
            pltpu.SemaphoreType.DMA((2, 2)),
            pltpu.SemaphoreType.DMA((N_DEV - 1,)),
            pltpu.SemaphoreType.DMA((N_DEV - 1,)),
            pltpu.SemaphoreType.DMA((N_DEV - 1,)),
            pltpu.SemaphoreType.DMA((N_DEV - 1,)),
        ],
        compiler_params=pltpu.CompilerParams(collective_id=0),
    )(xm, Wq, Wo, K, V)
    return out.reshape(1, SQ, D)


# device time: 51991 ns/iter; 1.5861x vs baseline; 1.5861x over previous
import jax
import jax.numpy as jnp
from jax import lax
from jax.experimental import pallas as pl
from jax.experimental.pallas import tpu as pltpu

N_DEV = 4
SQ = 512
SKV = 2048
D = 1024
H_GLOBAL = 32
H_PER = 8
DH = 128
SCALE = 0.08838834764831843
CHUNK = SQ // N_DEV


def kernel(x, Wq, Wo, K_ext, V_ext):
    xm = x.reshape(SQ, D)
    K = K_ext.reshape(SKV, H_GLOBAL, DH)
    V = V_ext.reshape(SKV, H_GLOBAL, DH)

    def body(x_ref, wq_ref, wo_ref, k_hbm, v_hbm, out_ref,
             q_ref, attn_ref, k_all, v_all, rs_buf,
             kv_sems, rs_send, rs_recv, ag_send, ag_recv):
        my = lax.axis_index("i")
        h0 = my * H_PER

        kv_copies = []
        for h in range(H_PER):
            kc = pltpu.make_async_copy(
                k_hbm.at[:, h0 + h, :], k_all.at[h], kv_sems.at[h, 0])
            vc = pltpu.make_async_copy(
                v_hbm.at[:, h0 + h, :], v_all.at[h], kv_sems.at[h, 1])
            kc.start()
            vc.start()
            kv_copies.append((kc, vc))

        q_ref[:, :] = jnp.dot(x_ref[:, :], wq_ref[:, :],
                              preferred_element_type=jnp.float32)

        barrier = pltpu.get_barrier_semaphore()
        for k in range(1, N_DEV):
            peer = lax.rem(my + k, N_DEV)
            pl.semaphore_signal(barrier, inc=1, device_id=(peer,),
                                device_id_type=pl.DeviceIdType.MESH)
        pl.semaphore_wait(barrier, N_DEV - 1)

        def compute_chunk(rows, first):
            for h in range(H_PER):
                if first:
                    kv_copies[h][0].wait()
                    kv_copies[h][1].wait()
                q_h = q_ref[rows, h * DH:(h + 1) * DH]
                s = lax.dot_general(
                    q_h, k_all[h],
                    (((1,), (1,)), ((), ())),
                    preferred_element_type=jnp.float32,
                ) * SCALE
                m = jnp.max(s, axis=1, keepdims=True)
                p = jnp.exp(s - m)
                l = jnp.sum(p, axis=1, keepdims=True)
                o = jnp.dot(p, v_all[h], preferred_element_type=jnp.float32)
                attn_ref[:, h * DH:(h + 1) * DH] = o / l
            return jnp.dot(attn_ref[:, :], wo_ref[:, :],
                           preferred_element_type=jnp.float32)

        rs_rdmas = []
        for k in range(1, N_DEV):
            c = lax.rem(my + k, N_DEV)
            rows = pl.ds(c * CHUNK, CHUNK)
            out_ref[rows, :] = compute_chunk(rows, first=(k == 1))
            slot = N_DEV - 1 - k
            rdma = pltpu.make_async_remote_copy(
                src_ref=out_ref.at[rows, :],
                dst_ref=rs_buf.at[slot],
                send_sem=rs_send.at[slot],
                recv_sem=rs_recv.at[slot],
                device_id=(c,),
                device_id_type=pl.DeviceIdType.MESH,
            )
            rdma.start()
            rs_rdmas.append(rdma)

        rows_my = pl.ds(my * CHUNK, CHUNK)
        y = compute_chunk(rows_my, first=False)
        for j in range(N_DEV - 1):
            recv = pltpu.make_async_remote_copy(
                src_ref=out_ref.at[rows_my, :],
                dst_ref=rs_buf.at[j],
                send_sem=rs_send.at[j],
                recv_sem=rs_recv.at[j],
                device_id=(my,),
                device_id_type=pl.DeviceIdType.MESH,
            )
            recv.wait_recv()
        out_ref[rows_my, :] = y + rs_buf[0] + rs_buf[1] + rs_buf[2]

        ag_rdmas = []
        for k in range(1, N_DEV):
            peer = lax.rem(my + k, N_DEV)
            slot = N_DEV - 1 - k
            rdma = pltpu.make_async_remote_copy(
                src_ref=out_ref.at[rows_my, :],
                dst_ref=out_ref.at[rows_my, :],
                send_sem=ag_send.at[slot],
                recv_sem=ag_recv.at[slot],
                device_id=(peer,),
                device_id_type=pl.DeviceIdType.MESH,
            )
            rdma.start()
            ag_rdmas.append(rdma)

        for j in range(N_DEV - 1):
            p = lax.rem(my + 1 + j, N_DEV)
            rows_p = pl.ds(p * CHUNK, CHUNK)
            recv = pltpu.make_async_remote_copy(
                src_ref=out_ref.at[rows_p, :],
                dst_ref=out_ref.at[rows_p, :],
                send_sem=ag_send.at[j],
                recv_sem=ag_recv.at[j],
                device_id=(p,),
                device_id_type=pl.DeviceIdType.MESH,
            )
            recv.wait_recv()

        for d in rs_rdmas + ag_rdmas:
            d.wait_send()

    out = pl.pallas_call(
        body,
        out_shape=jax.ShapeDtypeStruct((SQ, D), jnp.float32),
        in_specs=[
            pl.BlockSpec(memory_space=pltpu.MemorySpace.VMEM),
            pl.BlockSpec(memory_space=pltpu.MemorySpace.VMEM),
            pl.BlockSpec(memory_space=pltpu.MemorySpace.VMEM),
            pl.BlockSpec(memory_space=pltpu.MemorySpace.HBM),
            pl.BlockSpec(memory_space=pltpu.MemorySpace.HBM),
        ],
        out_specs=pl.BlockSpec(memory_space=pltpu.MemorySpace.VMEM),
        scratch_shapes=[
            pltpu.VMEM((SQ, D), jnp.float32),
            pltpu.VMEM((CHUNK, D), jnp.float32),
            pltpu.VMEM((H_PER, SKV, DH), jnp.float32),
            pltpu.VMEM((H_PER, SKV, DH), jnp.float32),
            pltpu.VMEM((N_DEV - 1, CHUNK, D), jnp.float32),
            pltpu.SemaphoreType.DMA((H_PER, 2)),
            pltpu.SemaphoreType.DMA((N_DEV - 1,)),
            pltpu.SemaphoreType.DMA((N_DEV - 1,)),
            pltpu.SemaphoreType.DMA((N_DEV - 1,)),
            pltpu.SemaphoreType.DMA((N_DEV - 1,)),
        ],
        compiler_params=pltpu.CompilerParams(collective_id=0),
    )(xm, Wq, Wo, K, V)
    return out.reshape(1, SQ, D)


# device time: 48137 ns/iter; 1.7131x vs baseline; 1.0801x over previous
import jax
import jax.numpy as jnp
from jax import lax
from jax.experimental import pallas as pl
from jax.experimental.pallas import tpu as pltpu

N_DEV = 4
SQ = 512
SKV = 2048
D = 1024
H_GLOBAL = 32
H_PER = 8
DH = 128
SCALE = 0.08838834764831843
CHUNK = SQ // N_DEV
BF16 = jnp.bfloat16


def kernel(x, Wq, Wo, K_ext, V_ext):
    xm = x.reshape(SQ, D)
    K = K_ext.reshape(SKV, H_GLOBAL, DH)
    V = V_ext.reshape(SKV, H_GLOBAL, DH)

    def body(x_ref, wq_ref, wo_ref, k_hbm, v_hbm, out_ref,
             q_bf, attn_bf, k_all, v_all, k_bf, v_bf, wo_b, rs_buf, ag_bf,
             kv_sems, rs_send, rs_recv, ag_send, ag_recv):
        my = lax.axis_index("i")
        h0 = my * H_PER

        kv_copies = []
        for h in range(H_PER):
            kc = pltpu.make_async_copy(
                k_hbm.at[:, h0 + h, :], k_all.at[h], kv_sems.at[h, 0])
            vc = pltpu.make_async_copy(
                v_hbm.at[:, h0 + h, :], v_all.at[h], kv_sems.at[h, 1])
            kc.start()
            vc.start()
            kv_copies.append((kc, vc))

        q_bf[:, :] = jnp.dot(
            x_ref[:, :].astype(BF16), wq_ref[:, :].astype(BF16),
            preferred_element_type=jnp.float32,
        ).astype(BF16)
        wo_b[:, :] = wo_ref[:, :].astype(BF16)

        barrier = pltpu.get_barrier_semaphore()
        for k in range(1, N_DEV):
            peer = lax.rem(my + k, N_DEV)
            pl.semaphore_signal(barrier, inc=1, device_id=(peer,),
                                device_id_type=pl.DeviceIdType.MESH)
        pl.semaphore_wait(barrier, N_DEV - 1)

        def compute_chunk(rows, first):
            for h in range(H_PER):
                if first:
                    kv_copies[h][0].wait()
                    kv_copies[h][1].wait()
                    k_bf[h] = k_all[h].astype(BF16)
                    v_bf[h] = v_all[h].astype(BF16)
                q_h = q_bf[rows, h * DH:(h + 1) * DH]
                s = lax.dot_general(
                    q_h, k_bf[h],
                    (((1,), (1,)), ((), ())),
                    preferred_element_type=jnp.float32,
                ) * SCALE
                m = jnp.max(s, axis=1, keepdims=True)
                p = jnp.exp(s - m)
                l = jnp.sum(p, axis=1, keepdims=True)
                o = jnp.dot(p.astype(BF16), v_bf[h],
                            preferred_element_type=jnp.float32)
                attn_bf[:, h * DH:(h + 1) * DH] = (o / l).astype(BF16)
            return jnp.dot(attn_bf[:, :], wo_b[:, :],
                           preferred_element_type=jnp.float32)

        rs_rdmas = []
        for k in range(1, N_DEV):
            c = lax.rem(my + k, N_DEV)
            rows = pl.ds(c * CHUNK, CHUNK)
            out_ref[rows, :] = compute_chunk(rows, first=(k == 1))
            slot = N_DEV - 1 - k
            rdma = pltpu.make_async_remote_copy(
                src_ref=out_ref.at[rows, :],
                dst_ref=rs_buf.at[slot],
                send_sem=rs_send.at[slot],
                recv_sem=rs_recv.at[slot],
                device_id=(c,),
                device_id_type=pl.DeviceIdType.MESH,
            )
            rdma.start()
            rs_rdmas.append(rdma)

        rows_my = pl.ds(my * CHUNK, CHUNK)
        y = compute_chunk(rows_my, first=False)
        for j in range(N_DEV - 1):
            recv = pltpu.make_async_remote_copy(
                src_ref=out_ref.at[rows_my, :],
                dst_ref=rs_buf.at[j],
                send_sem=rs_send.at[j],
                recv_sem=rs_recv.at[j],
                device_id=(my,),
                device_id_type=pl.DeviceIdType.MESH,
            )
            recv.wait_recv()
        y = y + rs_buf[0] + rs_buf[1] + rs_buf[2]
        out_ref[rows_my, :] = y
        ag_bf[my] = y.astype(BF16)

        ag_rdmas = []
        for k in range(1, N_DEV):
            peer = lax.rem(my + k, N_DEV)
            slot = N_DEV - 1 - k
            rdma = pltpu.make_async_remote_copy(
                src_ref=ag_bf.at[my],
                dst_ref=ag_bf.at[my],
                send_sem=ag_send.at[slot],
                recv_sem=ag_recv.at[slot],
                device_id=(peer,),
                device_id_type=pl.DeviceIdType.MESH,
            )
            rdma.start()
            ag_rdmas.append(rdma)

        for j in range(N_DEV - 1):
            p = lax.rem(my + 1 + j, N_DEV)
            recv = pltpu.make_async_remote_copy(
                src_ref=ag_bf.at[p],
                dst_ref=ag_bf.at[p],
                send_sem=ag_send.at[j],
                recv_sem=ag_recv.at[j],
                device_id=(p,),
                device_id_type=pl.DeviceIdType.MESH,
            )
            recv.wait_recv()
            out_ref[pl.ds(p * CHUNK, CHUNK), :] = ag_bf[p].astype(jnp.float32)

        for d in rs_rdmas + ag_rdmas:
            d.wait_send()

    out = pl.pallas_call(
        body,
        out_shape=jax.ShapeDtypeStruct((SQ, D), jnp.float32),
        in_specs=[
            pl.BlockSpec(memory_space=pltpu.MemorySpace.VMEM),
            pl.BlockSpec(memory_space=pltpu.MemorySpace.VMEM),
            pl.BlockSpec(memory_space=pltpu.MemorySpace.VMEM),
            pl.BlockSpec(memory_space=pltpu.MemorySpace.HBM),
            pl.BlockSpec(memory_space=pltpu.MemorySpace.HBM),
        ],
        out_specs=pl.BlockSpec(memory_space=pltpu.MemorySpace.VMEM),
        scratch_shapes=[
            pltpu.VMEM((SQ, D), BF16),
            pltpu.VMEM((CHUNK, D), BF16),
            pltpu.VMEM((H_PER, SKV, DH), jnp.float32),
            pltpu.VMEM((H_PER, SKV, DH), jnp.float32),
            pltpu.VMEM((H_PER, SKV, DH), BF16),
            pltpu.VMEM((H_PER, SKV, DH), BF16),
            pltpu.VMEM((D, D), BF16),
            pltpu.VMEM((N_DEV - 1, CHUNK, D), jnp.float32),
            pltpu.VMEM((N_DEV, CHUNK, D), BF16),
            pltpu.SemaphoreType.DMA((H_PER, 2)),
            pltpu.SemaphoreType.DMA((N_DEV - 1,)),
            pltpu.SemaphoreType.DMA((N_DEV - 1,)),
            pltpu.SemaphoreType.DMA((N_DEV - 1,)),
            pltpu.SemaphoreType.DMA((N_DEV - 1,)),
        ],
        compiler_params=pltpu.CompilerParams(
            collective_id=0,
            vmem_limit_bytes=100 * 1024 * 1024,
        ),
    )(xm, Wq, Wo, K, V)
    return out.reshape(1, SQ, D)
